# Initial kernel scaffold; baseline (speedup 1.0000x reference)
import jax
import jax.numpy as jnp
from jax.experimental import pallas as pl


def kernel(indices, embedding_weight):
    raise NotImplementedError("write your pallas kernel here")


# speedup vs baseline: 3.1869x; 3.1869x over previous
import functools

import jax
import jax.numpy as jnp
from jax import lax
from jax.experimental import pallas as pl
from jax.experimental.pallas import tpu as pltpu
from jax.experimental.pallas import tpu_sc as plsc

NUM_CODES = 8192
CODE_DIM = 256
B = 32
T = 1024

_NC = 2
_NS = 16
_NW = _NC * _NS

_TOTAL = B * T
_PER_W = _TOTAL // _NW
_CHUNK = 128
_NCHUNK = _PER_W // _CHUNK


def _gather_kernel(idx_hbm, table_hbm, out_hbm, idx_v, rows0, rows1, sem0, sem1):
    wid = lax.axis_index("s") * _NC + lax.axis_index("c")
    base = wid * _PER_W

    pltpu.sync_copy(idx_hbm.at[pl.ds(wid * _NCHUNK, _NCHUNK)], idx_v)

    bufs = (rows0, rows1)
    sems = (sem0, sem1)

    copies = [None, None]
    copies[0] = pltpu.async_copy(table_hbm.at[idx_v.at[0]], bufs[0], sems[0])

    for j in range(_NCHUNK):
        cur = j % 2
        nxt = (j + 1) % 2
        if j + 1 < _NCHUNK:
            copies[nxt] = pltpu.async_copy(
                table_hbm.at[idx_v.at[j + 1]], bufs[nxt], sems[nxt]
            )
        copies[cur].wait()
        pltpu.sync_copy(
            bufs[cur], out_hbm.at[pl.ds(base + j * _CHUNK, _CHUNK)]
        )


@jax.jit
def _gather(indices_2d, embedding_weight):
    mesh = plsc.VectorSubcoreMesh(core_axis_name="c", subcore_axis_name="s")
    run = functools.partial(
        pl.kernel,
        mesh=mesh,
        out_type=jax.ShapeDtypeStruct((_TOTAL, CODE_DIM), jnp.float32),
        scratch_types=[
            pltpu.VMEM((_NCHUNK, _CHUNK), jnp.int32),
            pltpu.VMEM((_CHUNK, CODE_DIM), jnp.float32),
            pltpu.VMEM((_CHUNK, CODE_DIM), jnp.float32),
            pltpu.SemaphoreType.DMA,
            pltpu.SemaphoreType.DMA,
        ],
    )(_gather_kernel)
    return run(indices_2d, embedding_weight)


def kernel(indices, embedding_weight):
    idx2d = indices.reshape(_TOTAL // _CHUNK, _CHUNK).astype(jnp.int32)
    out = _gather(idx2d, embedding_weight)
    return out.reshape(B, T, CODE_DIM)


# speedup vs baseline: 3.2071x; 1.0063x over previous
import functools

import jax
import jax.numpy as jnp
from jax import lax
from jax.experimental import pallas as pl
from jax.experimental.pallas import tpu as pltpu
from jax.experimental.pallas import tpu_sc as plsc

NUM_CODES = 8192
CODE_DIM = 256
B = 32
T = 1024

_NC = 2
_NS = 16
_NW = _NC * _NS

_TOTAL = B * T
_PER_W = _TOTAL // _NW
_CHUNK = 128
_NCHUNK = _PER_W // _CHUNK


_NBUF = 3


def _gather_kernel(idx_hbm, table_hbm, out_hbm, idx_v,
                   rows0, rows1, rows2, g0, g1, g2, w0, w1, w2):
    wid = lax.axis_index("s") * _NC + lax.axis_index("c")
    base = wid * _PER_W

    pltpu.sync_copy(idx_hbm.at[pl.ds(wid * _NCHUNK, _NCHUNK)], idx_v)

    bufs = (rows0, rows1, rows2)
    gsem = (g0, g1, g2)
    wsem = (w0, w1, w2)

    gathers = [None] * _NCHUNK
    writes = [None] * _NCHUNK

    def start_gather(j):
        gathers[j] = pltpu.async_copy(
            table_hbm.at[idx_v.at[j]], bufs[j % _NBUF], gsem[j % _NBUF]
        )

    for j in range(_NBUF - 1):
        start_gather(j)

    for j in range(_NCHUNK):
        gathers[j].wait()
        writes[j] = pltpu.async_copy(
            bufs[j % _NBUF],
            out_hbm.at[pl.ds(base + j * _CHUNK, _CHUNK)],
            wsem[j % _NBUF],
        )
        nxt = j + _NBUF - 1
        if nxt < _NCHUNK:
            if nxt - _NBUF >= 0:
                writes[nxt - _NBUF].wait()
            start_gather(nxt)

    for j in range(_NCHUNK - _NBUF, _NCHUNK):
        if j >= 0:
            writes[j].wait()


@jax.jit
def _gather(indices_2d, embedding_weight):
    mesh = plsc.VectorSubcoreMesh(core_axis_name="c", subcore_axis_name="s")
    run = functools.partial(
        pl.kernel,
        mesh=mesh,
        out_type=jax.ShapeDtypeStruct((_TOTAL, CODE_DIM), jnp.float32),
        scratch_types=[
            pltpu.VMEM((_NCHUNK, _CHUNK), jnp.int32),
            pltpu.VMEM((_CHUNK, CODE_DIM), jnp.float32),
            pltpu.VMEM((_CHUNK, CODE_DIM), jnp.float32),
            pltpu.VMEM((_CHUNK, CODE_DIM), jnp.float32),
            pltpu.SemaphoreType.DMA,
            pltpu.SemaphoreType.DMA,
            pltpu.SemaphoreType.DMA,
            pltpu.SemaphoreType.DMA,
            pltpu.SemaphoreType.DMA,
            pltpu.SemaphoreType.DMA,
        ],
    )(_gather_kernel)
    return run(indices_2d, embedding_weight)


def kernel(indices, embedding_weight):
    idx2d = indices.reshape(_TOTAL // _CHUNK, _CHUNK).astype(jnp.int32)
    out = _gather(idx2d, embedding_weight)
    return out.reshape(B, T, CODE_DIM)
